# Initial kernel scaffold; baseline (speedup 1.0000x reference)
import jax
import jax.numpy as jnp
from jax.experimental import pallas as pl


def kernel(vectors, node_feats, node_species, radial_embedding, receivers, species_embed, W_radial, W_vec, W_msg, W_resid, W_sc, W_read):
    raise NotImplementedError("write your pallas kernel here")


# speedup vs baseline: 2.8397x; 2.8397x over previous
import functools

import jax
import jax.numpy as jnp
from jax import lax
from jax.experimental import pallas as pl
from jax.experimental.pallas import tpu as pltpu
from jax.experimental.pallas import tpu_sc as plsc

N_NODES = 10000
N_EDGES = 320000
D = 128
R = 8
S = 10
EMB = 64
OUT = 16
AVG_DEG = 32.0

N_GROUPS = 4
N_SHARDS = 8
EDGES_PER_SHARD = N_EDGES // N_SHARDS
W_EDGES = 2000
N_WINDOWS = EDGES_PER_SHARD // W_EDGES
BLKE = 16
NCOMP = 11
ACC_W = N_NODES * R


@functools.cache
def _get_edge_kernel():
    mesh = plsc.VectorSubcoreMesh(core_axis_name="c", subcore_axis_name="s")
    return functools.partial(
        pl.kernel,
        out_type=jax.ShapeDtypeStruct((32 * ACC_W,), jnp.float32),
        mesh=mesh,
        scratch_types=[
            pltpu.VMEM((NCOMP * W_EDGES,), jnp.float32),
            pltpu.VMEM((W_EDGES,), jnp.int32),
            pltpu.VMEM((W_EDGES,), jnp.float32),
            pltpu.VMEM((ACC_W,), jnp.float32),
        ],
        compiler_params=pltpu.CompilerParams(needs_layout_passes=False),
    )(_edge_body)


def _edge_body(vx_h, vy_h, vz_h, r0_h, r1_h, r2_h, r3_h, r4_h, r5_h, r6_h,
               r7_h, recv_hbm, out_hbm, comp_v, recv_v, uk_v, acc_v):
    cid = lax.axis_index("c")
    sid = lax.axis_index("s")
    flat = cid * 16 + sid
    grp = flat // N_SHARDS
    shard = flat % N_SHARDS
    comp_hbm = (vx_h, vy_h, vz_h, r0_h, r1_h, r2_h, r3_h, r4_h, r5_h, r6_h,
                r7_h)

    zero16 = jnp.zeros((BLKE,), jnp.float32)

    def _zstore(i, c):
        acc_v[pl.ds(i * BLKE, BLKE)] = zero16
        return c
    lax.fori_loop(0, ACC_W // BLKE, _zstore, 0)

    lanes = lax.iota(jnp.int32, BLKE)
    cj8 = lanes % R
    half = lanes // R
    tmpl_re = (3 + cj8) * W_EDGES + half
    mask_lo = half == 0
    mask_hi = half == 1
    is_kgrp = grp > 0
    kci = jnp.where(is_kgrp, grp - 1, 0)

    def _window(w, carry):
        ebase = shard * EDGES_PER_SHARD + w * W_EDGES
        for ci in range(NCOMP):
            pltpu.sync_copy(comp_hbm[ci].at[pl.ds(ebase, W_EDGES)],
                            comp_v.at[pl.ds(ci * W_EDGES, W_EDGES)])
        pltpu.sync_copy(recv_hbm.at[pl.ds(ebase, W_EDGES)], recv_v)

        def _ublock(b, c):
            e0 = b * BLKE
            vx = comp_v[pl.ds(e0, BLKE)]
            vy = comp_v[pl.ds(W_EDGES + e0, BLKE)]
            vz = comp_v[pl.ds(2 * W_EDGES + e0, BLKE)]
            s = vx * vx + vy * vy + vz * vz
            si = plsc.bitcast(s, jnp.int32)
            seed = jnp.int32(0x5F3759DF) - lax.shift_right_logical(si, 1)
            y = plsc.bitcast(seed, jnp.float32)
            for _ in range(3):
                y = y * (1.5 - 0.5 * s * y * y)
            vk = jnp.where(kci == 0, vx, jnp.where(kci == 1, vy, vz))
            uk_v[pl.ds(e0, BLKE)] = vk * y
            return c

        def _pairs(p, c):
            e = 2 * p
            re_pair = plsc.load_gather(comp_v, [tmpl_re + e])
            racc = plsc.load_gather(recv_v, [half + e])
            u_pair = plsc.load_gather(uk_v, [half + e])
            val = jnp.where(is_kgrp, re_pair * u_pair, re_pair)
            idx = racc * R + cj8
            plsc.addupdate_scatter(acc_v, [idx], val, mask=mask_lo)
            plsc.addupdate_scatter(acc_v, [idx], val, mask=mask_hi)
            return c

        lax.fori_loop(0, W_EDGES // BLKE, _ublock, 0)
        lax.fori_loop(0, W_EDGES // 2, _pairs, 0)
        return carry

    lax.fori_loop(0, N_WINDOWS, _window, 0)

    pltpu.sync_copy(acc_v, out_hbm.at[pl.ds(flat * ACC_W, ACC_W)])


BLK = 1000
GRID_N = N_NODES // BLK
_PREC = lax.Precision.HIGHEST


def _dense_body(nf_ref, part_ref, spc_ref, wr_ref, wv_ref, wm_ref, wres_ref,
                wsc_ref, wread_ref, semb_ref, x_ref, ro_ref):
    wr = wr_ref[...]
    wv = wv_ref[...]
    a = jnp.dot(part_ref[0], wr, precision=_PREC)
    b = jnp.dot(part_ref[1], wv[0:1, :] * wr, precision=_PREC)
    b += jnp.dot(part_ref[2], wv[1:2, :] * wr, precision=_PREC)
    b += jnp.dot(part_ref[3], wv[2:3, :] * wr, precision=_PREC)
    nf = nf_ref[...]
    agg = nf * a + b
    pre = (jnp.dot(agg * (1.0 / AVG_DEG), wm_ref[...], precision=_PREC)
           + jnp.dot(nf, wres_ref[...], precision=_PREC))
    spc = spc_ref[0]
    sp_iota = lax.broadcasted_iota(jnp.int32, (S, BLK), 0)
    onehot_t = (jnp.broadcast_to(spc, (S, BLK)) == sp_iota).astype(jnp.float32)
    table2 = jnp.dot(semb_ref[...], wsc_ref[...], precision=_PREC)
    g = lax.dot_general(onehot_t, table2, (((0,), (0,)), ((), ())),
                        precision=_PREC)
    x = pre * (g * jax.nn.sigmoid(g))
    x_ref[...] = x
    ro_ref[...] = jnp.dot(x, wread_ref[...], precision=_PREC)


_dense_call = pl.pallas_call(
    _dense_body,
    grid=(GRID_N,),
    in_specs=[
        pl.BlockSpec((BLK, D), lambda i: (i, 0)),
        pl.BlockSpec((N_GROUPS, BLK, R), lambda i: (0, i, 0)),
        pl.BlockSpec((1, 1, BLK), lambda i: (i, 0, 0)),
        pl.BlockSpec((R, D), lambda i: (0, 0)),
        pl.BlockSpec((3, D), lambda i: (0, 0)),
        pl.BlockSpec((D, D), lambda i: (0, 0)),
        pl.BlockSpec((D, D), lambda i: (0, 0)),
        pl.BlockSpec((EMB, D), lambda i: (0, 0)),
        pl.BlockSpec((D, OUT), lambda i: (0, 0)),
        pl.BlockSpec((S, EMB), lambda i: (0, 0)),
    ],
    out_specs=[
        pl.BlockSpec((BLK, D), lambda i: (i, 0)),
        pl.BlockSpec((BLK, OUT), lambda i: (i, 0)),
    ],
    out_shape=[
        jax.ShapeDtypeStruct((N_NODES, D), jnp.float32),
        jax.ShapeDtypeStruct((N_NODES, OUT), jnp.float32),
    ],
)


@jax.jit
def kernel(vectors, node_feats, node_species, radial_embedding, receivers,
           species_embed, W_radial, W_vec, W_msg, W_resid, W_sc, W_read):
    comps = [vectors[:, k] for k in range(3)]
    comps += [radial_embedding[:, j] for j in range(R)]
    recv1 = receivers.astype(jnp.int32)
    partials = _get_edge_kernel()(*comps, recv1)
    seg = partials.reshape(N_GROUPS, N_SHARDS, N_NODES, R).sum(axis=1)
    spc3 = node_species.astype(jnp.int32).reshape(GRID_N, 1, BLK)
    x, readout = _dense_call(node_feats, seg, spc3, W_radial, W_vec,
                             W_msg, W_resid, W_sc, W_read, species_embed)
    return (x, readout)
